# baseline (speedup 1.0000x reference)
import functools

import jax
import jax.numpy as jnp
from jax import lax
from jax.experimental import pallas as pl
from jax.experimental.pallas import tpu as pltpu
from jax.experimental.pallas import tpu_sc as plsc

N = 100000
E = 1600000
HID = 16
F = 16
KR = 5
WIN = KR * 128
NWIN = E // WIN
NS = 16
NC = 2
STRIPE = 6256
ACC_N = NS * STRIPE
LAST = N - (NS - 1) * STRIPE
ZROWS = 368

NB = 2048
GRID = (N + NB - 1) // NB

_GDN = lax.GatherDimensionNumbers(
    offset_dims=(), collapsed_slice_dims=(0,), start_index_map=(0,))


def _bcast_lane(v16, j):
    idx = jnp.full((16, 1), j, jnp.int32)
    return lax.gather(v16, idx, _GDN, (1,),
                      mode=lax.GatherScatterMode.PROMISE_IN_BOUNDS)


def _sc_cheb_body(x0_hbm, cols_hbm, rows_hbm, vals_hbm, y1_hbm, y2_hbm,
                  cols_a, rows_a, vals_a, cols_b, rows_b, vals_b,
                  gbuf_a, gbuf_b, zbuf, acc,
                  sem_i, sem_ga, sem_gb, sem_sa, sem_sb):
    c = lax.axis_index("c")
    s = lax.axis_index("s")
    c_n = (c * N).astype(jnp.int32)
    base = s * STRIPE

    def zero_stripe():
        for k in range(STRIPE // ZROWS):
            pltpu.sync_copy(zbuf, acc.at[pl.ds(base + k * ZROWS, ZROWS), :])

    def zfill(i, carry):
        zbuf[i, :] = jnp.zeros((16,), jnp.float32)
        return carry
    lax.fori_loop(0, ZROWS, zfill, 0)
    zero_stripe()
    plsc.subcore_barrier()

    trips = (NWIN - s + NS - 1) // NS

    def load_idx(i, cols_v, rows_v, vals_v):
        w = i * NS + s
        d1 = pltpu.async_copy(cols_hbm.at[w], cols_v, sem_i)
        d2 = pltpu.async_copy(rows_hbm.at[w], rows_v, sem_i)
        d3 = pltpu.async_copy(vals_hbm.at[w], vals_v, sem_i)
        d1.wait()
        d2.wait()
        d3.wait()

        def adj(g, carry2):
            jj = g // 8
            off = (g % 8) * 16
            cols_v[jj, pl.ds(off, 16)] = cols_v[jj, pl.ds(off, 16)] + c_n
            return carry2
        lax.fori_loop(0, KR * 8, adj, 0)

    def scale(gbuf, vals_v):
        def mulg(g, carry2):
            jj = g // 8
            off = (g % 8) * 16
            v16 = vals_v[jj, pl.ds(off, 16)]
            e0 = g * 16
            for j in range(16):
                bv = _bcast_lane(v16, j)
                gbuf[e0 + j, :] = gbuf[e0 + j, :] * bv
            return carry2
        lax.fori_loop(0, KR * 8, mulg, 0)

    def wait_scatters(gbuf, rows_v, sem):
        for j in range(KR):
            pltpu.make_async_copy(gbuf.at[pl.ds(j * 128, 128), :],
                                  acc.at[rows_v.at[j]], sem).wait()

    def lap_apply(src_tbl, dst_hbm):

        def start_gathers(cols_v, gbuf, sem):
            for j in range(KR):
                pltpu.async_copy(src_tbl.at[cols_v.at[j]],
                                 gbuf.at[pl.ds(j * 128, 128), :], sem)

        def wait_gathers(cols_v, gbuf, sem):
            for j in range(KR):
                pltpu.make_async_copy(src_tbl.at[cols_v.at[j]],
                                      gbuf.at[pl.ds(j * 128, 128), :],
                                      sem).wait()

        def start_scatters(gbuf, rows_v, sem):
            for j in range(KR):
                pltpu.async_copy(gbuf.at[pl.ds(j * 128, 128), :],
                                 acc.at[rows_v.at[j]], sem, add=True)

        bufs_a = (cols_a, rows_a, vals_a, gbuf_a, sem_ga, sem_sa)
        bufs_b = (cols_b, rows_b, vals_b, gbuf_b, sem_gb, sem_sb)

        def phase(i, bx, by):
            cols_x, rows_x, vals_x, gbuf_x, sem_gx, sem_sx = bx
            cols_y, rows_y, vals_y, gbuf_y, sem_gy, sem_sy = by

            @pl.when(i < trips)
            def _run():
                @pl.when(i >= 1)
                def _free_y():
                    wait_scatters(gbuf_y, rows_y, sem_sy)

                @pl.when(i + 1 < trips)
                def _stage_next():
                    load_idx(i + 1, cols_y, rows_y, vals_y)

                wait_gathers(cols_x, gbuf_x, sem_gx)

                @pl.when(i + 1 < trips)
                def _prefetch_next():
                    start_gathers(cols_y, gbuf_y, sem_gy)

                scale(gbuf_x, vals_x)
                start_scatters(gbuf_x, rows_x, sem_sx)

        load_idx(0, cols_a, rows_a, vals_a)
        start_gathers(cols_a, gbuf_a, sem_ga)

        def step(t, carry):
            phase(2 * t, bufs_a, bufs_b)
            phase(2 * t + 1, bufs_b, bufs_a)
            return carry

        lax.fori_loop(0, (trips + 1) // 2, step, 0)

        @pl.when(((trips - 1) % 2) == 0)
        def _drain_a():
            wait_scatters(gbuf_a, rows_a, sem_sa)

        @pl.when(((trips - 1) % 2) == 1)
        def _drain_b():
            wait_scatters(gbuf_b, rows_b, sem_sb)

        plsc.subcore_barrier()

        @pl.when(s < NS - 1)
        def _full_stripe():
            pltpu.sync_copy(acc.at[pl.ds(base, STRIPE), :],
                            dst_hbm.at[pl.ds(c_n + base, STRIPE), :])

        @pl.when(s == NS - 1)
        def _last_stripe():
            pltpu.sync_copy(acc.at[pl.ds(base, LAST), :],
                            dst_hbm.at[pl.ds(c_n + base, LAST), :])

    lap_apply(x0_hbm, y1_hbm)
    zero_stripe()
    plsc.subcore_barrier()
    lap_apply(y1_hbm, y2_hbm)


@functools.cache
def _sc_cheb():
  return pl.kernel(
    _sc_cheb_body,
    out_type=[jax.ShapeDtypeStruct((NC * N, F), jnp.float32),
              jax.ShapeDtypeStruct((NC * N, F), jnp.float32)],
    mesh=plsc.VectorSubcoreMesh(core_axis_name="c", subcore_axis_name="s",
                                num_cores=NC, num_subcores=NS),
    compiler_params=pltpu.CompilerParams(use_tc_tiling_on_sc=False),
    scratch_types=[
        pltpu.VMEM((KR, 128), jnp.int32),
        pltpu.VMEM((KR, 128), jnp.int32),
        pltpu.VMEM((KR, 128), jnp.float32),
        pltpu.VMEM((KR, 128), jnp.int32),
        pltpu.VMEM((KR, 128), jnp.int32),
        pltpu.VMEM((KR, 128), jnp.float32),
        pltpu.VMEM((WIN, F), jnp.float32),
        pltpu.VMEM((WIN, F), jnp.float32),
        pltpu.VMEM((ZROWS, F), jnp.float32),
        pltpu.VMEM_SHARED((ACC_N, F), jnp.float32),
        pltpu.SemaphoreType.DMA,
        pltpu.SemaphoreType.DMA,
        pltpu.SemaphoreType.DMA,
        pltpu.SemaphoreType.DMA,
        pltpu.SemaphoreType.DMA,
    ],
  )


def _prep_body(inp_ref, h_ref, x0_ref):
    x0_ref[0] = inp_ref[...].T
    x0_ref[1] = h_ref[...].T


_tc_prep = pl.pallas_call(
    _prep_body,
    grid=(GRID,),
    in_specs=[
        pl.BlockSpec((HID, NB), lambda i: (0, i)),
        pl.BlockSpec((HID, NB), lambda i: (0, i)),
    ],
    out_specs=pl.BlockSpec((NC, NB, F), lambda i: (0, i, 0)),
    out_shape=jax.ShapeDtypeStruct((NC, N, F), jnp.float32),
)


def _tc_body(x0_ref, y1_ref, y2_ref, w_ref, b_ref, c_ref,
             wci_ref, wcf_ref, wco_ref, h_ref, cn_ref):
    a = jnp.concatenate(
        [x0_ref[0], x0_ref[1], y1_ref[0], y1_ref[1], y2_ref[0], y2_ref[1]],
        axis=1)
    w0, w1, w2 = w_ref[0], w_ref[1], w_ref[2]
    weff = jnp.concatenate([w0 - w2, w1, 2.0 * w2], axis=0)
    conv_t = lax.dot_general(weff, a, (((0,), (1,)), ((), ())),
                             preferred_element_type=jnp.float32)
    conv_t = conv_t + b_ref[0][:, None]
    cc = c_ref[...]
    ig = jax.nn.sigmoid(conv_t[0:16] + wci_ref[...] * cc)
    fg = jax.nn.sigmoid(conv_t[16:32] + wcf_ref[...] * cc)
    c_new = fg * cc + ig * jnp.tanh(conv_t[32:48])
    og = jax.nn.sigmoid(conv_t[48:64] + wco_ref[...] * c_new)
    h_ref[...] = og * jnp.tanh(c_new)
    cn_ref[...] = c_new


_tc_gates = pl.pallas_call(
    _tc_body,
    grid=(GRID,),
    in_specs=[
        pl.BlockSpec((NC, NB, F), lambda i: (0, i, 0)),
        pl.BlockSpec((NC, NB, F), lambda i: (0, i, 0)),
        pl.BlockSpec((NC, NB, F), lambda i: (0, i, 0)),
        pl.BlockSpec((3, 32, 64), lambda i: (0, 0, 0)),
        pl.BlockSpec((1, 64), lambda i: (0, 0)),
        pl.BlockSpec((HID, NB), lambda i: (0, i)),
        pl.BlockSpec((HID, NB), lambda i: (0, i)),
        pl.BlockSpec((HID, NB), lambda i: (0, i)),
        pl.BlockSpec((HID, NB), lambda i: (0, i)),
    ],
    out_specs=[
        pl.BlockSpec((HID, NB), lambda i: (0, i)),
        pl.BlockSpec((HID, NB), lambda i: (0, i)),
    ],
    out_shape=[
        jax.ShapeDtypeStruct((HID, N), jnp.float32),
        jax.ShapeDtypeStruct((HID, N), jnp.float32),
    ],
)


def kernel(input_tensor, h_cur, c_cur, lap_rows, lap_cols, lap_vals,
           W_cheb, b, W_ci, W_cf, W_co):
    x0 = _tc_prep(input_tensor[0], h_cur[0])
    cols3 = lap_cols.reshape(NWIN, KR, 128)
    rows3 = lap_rows.reshape(NWIN, KR, 128)
    vals3 = lap_vals.reshape(NWIN, KR, 128)

    y1, y2 = _sc_cheb()(x0.reshape(NC * N, F), cols3, rows3, vals3)

    h_new, c_new = _tc_gates(
        x0, y1.reshape(NC, N, F), y2.reshape(NC, N, F),
        W_cheb, b.reshape(1, 64), c_cur[0], W_ci, W_cf, W_co)
    return h_new[None], c_new[None]


# speedup vs baseline: 1.0194x; 1.0194x over previous
import functools

import jax
import jax.numpy as jnp
from jax import lax
from jax.experimental import pallas as pl
from jax.experimental.pallas import tpu as pltpu
from jax.experimental.pallas import tpu_sc as plsc

N = 100000
E = 1600000
HID = 16
F = 16
KR = 5
WIN = KR * 128
NWIN = E // WIN
NS = 16
NC = 2
STRIPE = 6256
ACC_N = NS * STRIPE
LAST = N - (NS - 1) * STRIPE
ZROWS = 368

NB = 2048
GRID = (N + NB - 1) // NB

_GDN = lax.GatherDimensionNumbers(
    offset_dims=(), collapsed_slice_dims=(0,), start_index_map=(0,))


def _bcast_lane(v16, j):
    idx = jnp.full((16, 1), j, jnp.int32)
    return lax.gather(v16, idx, _GDN, (1,),
                      mode=lax.GatherScatterMode.PROMISE_IN_BOUNDS)


def _sc_cheb_body(x0_hbm, cols_hbm, rows_hbm, vals_hbm, y1_hbm, y2_hbm,
                  cols_a, rows_a, vals_a, cols_b, rows_b, vals_b,
                  gbuf_a, gbuf_b, zbuf, acc,
                  sem_i, sem_ga, sem_gb, sem_sa, sem_sb):
    c = lax.axis_index("c")
    s = lax.axis_index("s")
    c_n = (c * N).astype(jnp.int32)
    base = s * STRIPE

    def zero_stripe():
        for k in range(STRIPE // ZROWS):
            pltpu.sync_copy(zbuf, acc.at[pl.ds(base + k * ZROWS, ZROWS), :])

    def zfill(i, carry):
        zbuf[i, :] = jnp.zeros((16,), jnp.float32)
        return carry
    lax.fori_loop(0, ZROWS, zfill, 0)
    zero_stripe()
    plsc.subcore_barrier()

    trips = (NWIN - s + NS - 1) // NS

    def load_idx(i, cols_v, rows_v, vals_v):
        w = i * NS + s
        d1 = pltpu.async_copy(cols_hbm.at[w], cols_v, sem_i)
        d2 = pltpu.async_copy(rows_hbm.at[w], rows_v, sem_i)
        d3 = pltpu.async_copy(vals_hbm.at[w], vals_v, sem_i)
        d1.wait()
        d2.wait()
        d3.wait()

        def adj(g, carry2):
            jj = g // 8
            off = (g % 8) * 16
            cols_v[jj, pl.ds(off, 16)] = cols_v[jj, pl.ds(off, 16)] + c_n
            return carry2
        lax.fori_loop(0, KR * 8, adj, 0)

    def scale(gbuf, vals_v):
        def mulg(g, carry2):
            jj = g // 8
            off = (g % 8) * 16
            v16 = vals_v[jj, pl.ds(off, 16)]
            e0 = g * 16
            for j in range(16):
                bv = _bcast_lane(v16, j)
                gbuf[e0 + j, :] = gbuf[e0 + j, :] * bv
            return carry2
        lax.fori_loop(0, KR * 8, mulg, 0)

    def wait_scatters(gbuf, rows_v, sem):
        for j in range(KR):
            pltpu.make_async_copy(gbuf.at[pl.ds(j * 128, 128), :],
                                  acc.at[rows_v.at[j]], sem).wait()

    def lap_apply(src_tbl, dst_hbm):

        def start_gathers(cols_v, gbuf, sem):
            for j in range(KR):
                pltpu.async_copy(src_tbl.at[cols_v.at[j]],
                                 gbuf.at[pl.ds(j * 128, 128), :], sem)

        def wait_gathers(cols_v, gbuf, sem):
            for j in range(KR):
                pltpu.make_async_copy(src_tbl.at[cols_v.at[j]],
                                      gbuf.at[pl.ds(j * 128, 128), :],
                                      sem).wait()

        def start_scatters(gbuf, rows_v, sem):
            for j in range(KR):
                pltpu.async_copy(gbuf.at[pl.ds(j * 128, 128), :],
                                 acc.at[rows_v.at[j]], sem, add=True)

        bufs_a = (cols_a, rows_a, vals_a, gbuf_a, sem_ga, sem_sa)
        bufs_b = (cols_b, rows_b, vals_b, gbuf_b, sem_gb, sem_sb)

        def phase(i, bx, by):
            cols_x, rows_x, vals_x, gbuf_x, sem_gx, sem_sx = bx
            cols_y, rows_y, vals_y, gbuf_y, sem_gy, sem_sy = by

            @pl.when(i < trips)
            def _run():
                @pl.when(i >= 1)
                def _free_y():
                    wait_scatters(gbuf_y, rows_y, sem_sy)

                @pl.when(i + 1 < trips)
                def _stage_next():
                    load_idx(i + 1, cols_y, rows_y, vals_y)

                wait_gathers(cols_x, gbuf_x, sem_gx)

                @pl.when(i + 1 < trips)
                def _prefetch_next():
                    start_gathers(cols_y, gbuf_y, sem_gy)

                scale(gbuf_x, vals_x)
                start_scatters(gbuf_x, rows_x, sem_sx)

        load_idx(0, cols_a, rows_a, vals_a)
        start_gathers(cols_a, gbuf_a, sem_ga)

        def step(t, carry):
            phase(2 * t, bufs_a, bufs_b)
            phase(2 * t + 1, bufs_b, bufs_a)
            return carry

        lax.fori_loop(0, (trips + 1) // 2, step, 0)

        @pl.when(((trips - 1) % 2) == 0)
        def _drain_a():
            wait_scatters(gbuf_a, rows_a, sem_sa)

        @pl.when(((trips - 1) % 2) == 1)
        def _drain_b():
            wait_scatters(gbuf_b, rows_b, sem_sb)

        plsc.subcore_barrier()

        @pl.when(s < NS - 1)
        def _full_stripe():
            pltpu.sync_copy(acc.at[pl.ds(base, STRIPE), :],
                            dst_hbm.at[pl.ds(c_n + base, STRIPE), :])

        @pl.when(s == NS - 1)
        def _last_stripe():
            pltpu.sync_copy(acc.at[pl.ds(base, LAST), :],
                            dst_hbm.at[pl.ds(c_n + base, LAST), :])

    lap_apply(x0_hbm, y1_hbm)
    zero_stripe()
    plsc.subcore_barrier()
    lap_apply(y1_hbm, y2_hbm)


@functools.cache
def _sc_cheb():
  return pl.kernel(
    _sc_cheb_body,
    out_type=[jax.ShapeDtypeStruct((NC * N, F), jnp.float32),
              jax.ShapeDtypeStruct((NC * N, F), jnp.float32)],
    mesh=plsc.VectorSubcoreMesh(core_axis_name="c", subcore_axis_name="s",
                                num_cores=NC, num_subcores=NS),
    compiler_params=pltpu.CompilerParams(use_tc_tiling_on_sc=False),
    scratch_types=[
        pltpu.VMEM((KR, 128), jnp.int32),
        pltpu.VMEM((KR, 128), jnp.int32),
        pltpu.VMEM((KR, 128), jnp.float32),
        pltpu.VMEM((KR, 128), jnp.int32),
        pltpu.VMEM((KR, 128), jnp.int32),
        pltpu.VMEM((KR, 128), jnp.float32),
        pltpu.VMEM((WIN, F), jnp.float32),
        pltpu.VMEM((WIN, F), jnp.float32),
        pltpu.VMEM((ZROWS, F), jnp.float32),
        pltpu.VMEM_SHARED((ACC_N, F), jnp.float32),
        pltpu.SemaphoreType.DMA,
        pltpu.SemaphoreType.DMA,
        pltpu.SemaphoreType.DMA,
        pltpu.SemaphoreType.DMA,
        pltpu.SemaphoreType.DMA,
    ],
  )


def _prep_body(inp_ref, h_ref, x0_ref):
    x0_ref[0] = inp_ref[...].T
    x0_ref[1] = h_ref[...].T


_tc_prep = pl.pallas_call(
    _prep_body,
    grid=(GRID,),
    in_specs=[
        pl.BlockSpec((HID, NB), lambda i: (0, i)),
        pl.BlockSpec((HID, NB), lambda i: (0, i)),
    ],
    out_specs=pl.BlockSpec((NC, NB, F), lambda i: (0, i, 0)),
    out_shape=jax.ShapeDtypeStruct((NC, N, F), jnp.float32),
)


def _tc_body(x0_ref, y1_ref, y2_ref, w_ref, b_ref, c_ref,
             wci_ref, wcf_ref, wco_ref, h_ref, cn_ref):
    a = jnp.concatenate(
        [x0_ref[0], x0_ref[1], y1_ref[0], y1_ref[1], y2_ref[0], y2_ref[1]],
        axis=1)
    w0, w1, w2 = w_ref[0], w_ref[1], w_ref[2]
    weff = jnp.concatenate([w0 - w2, w1, 2.0 * w2], axis=0)
    conv_t = lax.dot_general(weff, a, (((0,), (1,)), ((), ())),
                             preferred_element_type=jnp.float32)
    conv_t = conv_t + b_ref[0][:, None]
    cc = c_ref[...]
    ig = jax.nn.sigmoid(conv_t[0:16] + wci_ref[...] * cc)
    fg = jax.nn.sigmoid(conv_t[16:32] + wcf_ref[...] * cc)
    c_new = fg * cc + ig * jnp.tanh(conv_t[32:48])
    og = jax.nn.sigmoid(conv_t[48:64] + wco_ref[...] * c_new)
    h_ref[...] = og * jnp.tanh(c_new)
    cn_ref[...] = c_new


_tc_gates = pl.pallas_call(
    _tc_body,
    grid=(GRID,),
    in_specs=[
        pl.BlockSpec((NC, NB, F), lambda i: (0, i, 0)),
        pl.BlockSpec((NC, NB, F), lambda i: (0, i, 0)),
        pl.BlockSpec((NC, NB, F), lambda i: (0, i, 0)),
        pl.BlockSpec((3, 32, 64), lambda i: (0, 0, 0)),
        pl.BlockSpec((1, 64), lambda i: (0, 0)),
        pl.BlockSpec((HID, NB), lambda i: (0, i)),
        pl.BlockSpec((HID, NB), lambda i: (0, i)),
        pl.BlockSpec((HID, NB), lambda i: (0, i)),
        pl.BlockSpec((HID, NB), lambda i: (0, i)),
    ],
    out_specs=[
        pl.BlockSpec((HID, NB), lambda i: (0, i)),
        pl.BlockSpec((HID, NB), lambda i: (0, i)),
    ],
    out_shape=[
        jax.ShapeDtypeStruct((HID, N), jnp.float32),
        jax.ShapeDtypeStruct((HID, N), jnp.float32),
    ],
)


def kernel(input_tensor, h_cur, c_cur, lap_rows, lap_cols, lap_vals,
           W_cheb, b, W_ci, W_cf, W_co):
    x0 = jnp.concatenate(
        [input_tensor[0].T, h_cur[0].T], axis=0)
    cols3 = lap_cols.reshape(NWIN, KR, 128)
    rows3 = lap_rows.reshape(NWIN, KR, 128)
    vals3 = lap_vals.reshape(NWIN, KR, 128)

    y1, y2 = _sc_cheb()(x0, cols3, rows3, vals3)

    h_new, c_new = _tc_gates(
        x0.reshape(NC, N, F), y1.reshape(NC, N, F), y2.reshape(NC, N, F),
        W_cheb, b.reshape(1, 64), c_cur[0], W_ci, W_cf, W_co)
    return h_new[None], c_new[None]


# speedup vs baseline: 1.0695x; 1.0492x over previous
import functools

import jax
import jax.numpy as jnp
from jax import lax
from jax.experimental import pallas as pl
from jax.experimental.pallas import tpu as pltpu
from jax.experimental.pallas import tpu_sc as plsc

N = 100000
E = 1600000
HID = 16
F = 16
KR = 5
WIN = KR * 128
NWIN = E // WIN
NS = 16
NC = 2
STRIPE = 6256
ACC_N = NS * STRIPE
LAST = N - (NS - 1) * STRIPE
ZROWS = 368

NB = 2048
GRID = (N + NB - 1) // NB

_GDN = lax.GatherDimensionNumbers(
    offset_dims=(), collapsed_slice_dims=(0,), start_index_map=(0,))


def _bcast_lane(v16, j):
    idx = jnp.full((16, 1), j, jnp.int32)
    return lax.gather(v16, idx, _GDN, (1,),
                      mode=lax.GatherScatterMode.PROMISE_IN_BOUNDS)


def _sc_lap_body(x_hbm, cols_hbm, rows_hbm, vals_hbm, out_hbm,
                 cols_a, rows_a, vals_a, cols_b, rows_b, vals_b,
                 gbuf_a, gbuf_b, zbuf, acc,
                 sem_i, sem_ga, sem_gb, sem_sa, sem_sb):
    c = lax.axis_index("c")
    s = lax.axis_index("s")
    c_n = (c * N).astype(jnp.int32)

    def zfill(i, carry):
        zbuf[i, :] = jnp.zeros((16,), jnp.float32)
        return carry
    lax.fori_loop(0, ZROWS, zfill, 0)
    base = s * STRIPE
    for k in range(STRIPE // ZROWS):
        pltpu.sync_copy(zbuf, acc.at[pl.ds(base + k * ZROWS, ZROWS), :])
    plsc.subcore_barrier()

    trips = (NWIN - s + NS - 1) // NS

    def load_idx(i, cols_v, rows_v, vals_v):
        w = i * NS + s
        d1 = pltpu.async_copy(cols_hbm.at[w], cols_v, sem_i)
        d2 = pltpu.async_copy(rows_hbm.at[w], rows_v, sem_i)
        d3 = pltpu.async_copy(vals_hbm.at[w], vals_v, sem_i)
        d1.wait()
        d2.wait()
        d3.wait()

        def adj(g, carry2):
            jj = g // 8
            off = (g % 8) * 16
            cols_v[jj, pl.ds(off, 16)] = cols_v[jj, pl.ds(off, 16)] + c_n
            return carry2
        lax.fori_loop(0, KR * 8, adj, 0)

    def start_gathers(cols_v, gbuf, sem):
        for j in range(KR):
            pltpu.async_copy(x_hbm.at[cols_v.at[j]],
                             gbuf.at[pl.ds(j * 128, 128), :], sem)

    def wait_gathers(cols_v, gbuf, sem):
        for j in range(KR):
            pltpu.make_async_copy(x_hbm.at[cols_v.at[j]],
                                  gbuf.at[pl.ds(j * 128, 128), :], sem).wait()

    def scale(gbuf, vals_v):
        def mulg(g, carry2):
            jj = g // 8
            off = (g % 8) * 16
            v16 = vals_v[jj, pl.ds(off, 16)]
            e0 = g * 16
            for j in range(16):
                bv = _bcast_lane(v16, j)
                gbuf[e0 + j, :] = gbuf[e0 + j, :] * bv
            return carry2
        lax.fori_loop(0, KR * 8, mulg, 0)

    def start_scatters(gbuf, rows_v, sem):
        for j in range(KR):
            pltpu.async_copy(gbuf.at[pl.ds(j * 128, 128), :],
                             acc.at[rows_v.at[j]], sem, add=True)

    def wait_scatters(gbuf, rows_v, sem):
        for j in range(KR):
            pltpu.make_async_copy(gbuf.at[pl.ds(j * 128, 128), :],
                                  acc.at[rows_v.at[j]], sem).wait()

    bufs_a = (cols_a, rows_a, vals_a, gbuf_a, sem_ga, sem_sa)
    bufs_b = (cols_b, rows_b, vals_b, gbuf_b, sem_gb, sem_sb)

    def phase(i, bx, by):
        cols_x, rows_x, vals_x, gbuf_x, sem_gx, sem_sx = bx
        cols_y, rows_y, vals_y, gbuf_y, sem_gy, sem_sy = by

        @pl.when(i < trips)
        def _run():
            @pl.when(i >= 1)
            def _free_y():
                wait_scatters(gbuf_y, rows_y, sem_sy)

            @pl.when(i + 1 < trips)
            def _stage_next():
                load_idx(i + 1, cols_y, rows_y, vals_y)

            wait_gathers(cols_x, gbuf_x, sem_gx)

            @pl.when(i + 1 < trips)
            def _prefetch_next():
                start_gathers(cols_y, gbuf_y, sem_gy)

            scale(gbuf_x, vals_x)
            start_scatters(gbuf_x, rows_x, sem_sx)

    load_idx(0, cols_a, rows_a, vals_a)
    start_gathers(cols_a, gbuf_a, sem_ga)

    def step(t, carry):
        phase(2 * t, bufs_a, bufs_b)
        phase(2 * t + 1, bufs_b, bufs_a)
        return carry

    lax.fori_loop(0, (trips + 1) // 2, step, 0)

    @pl.when(((trips - 1) % 2) == 0)
    def _drain_a():
        wait_scatters(gbuf_a, rows_a, sem_sa)

    @pl.when(((trips - 1) % 2) == 1)
    def _drain_b():
        wait_scatters(gbuf_b, rows_b, sem_sb)

    plsc.subcore_barrier()

    @pl.when(s < NS - 1)
    def _full_stripe():
        pltpu.sync_copy(acc.at[pl.ds(base, STRIPE), :],
                        out_hbm.at[pl.ds(c_n + base, STRIPE), :])

    @pl.when(s == NS - 1)
    def _last_stripe():
        pltpu.sync_copy(acc.at[pl.ds(base, LAST), :],
                        out_hbm.at[pl.ds(c_n + base, LAST), :])


@functools.cache
def _sc_lap():
  return pl.kernel(
    _sc_lap_body,
    out_type=jax.ShapeDtypeStruct((NC * N, F), jnp.float32),
    mesh=plsc.VectorSubcoreMesh(core_axis_name="c", subcore_axis_name="s",
                                num_cores=NC, num_subcores=NS),
    compiler_params=pltpu.CompilerParams(use_tc_tiling_on_sc=False),
    scratch_types=[
        pltpu.VMEM((KR, 128), jnp.int32),
        pltpu.VMEM((KR, 128), jnp.int32),
        pltpu.VMEM((KR, 128), jnp.float32),
        pltpu.VMEM((KR, 128), jnp.int32),
        pltpu.VMEM((KR, 128), jnp.int32),
        pltpu.VMEM((KR, 128), jnp.float32),
        pltpu.VMEM((WIN, F), jnp.float32),
        pltpu.VMEM((WIN, F), jnp.float32),
        pltpu.VMEM((ZROWS, F), jnp.float32),
        pltpu.VMEM_SHARED((ACC_N, F), jnp.float32),
        pltpu.SemaphoreType.DMA,
        pltpu.SemaphoreType.DMA,
        pltpu.SemaphoreType.DMA,
        pltpu.SemaphoreType.DMA,
        pltpu.SemaphoreType.DMA,
    ],
  )


def _tc_body(x0_ref, y1_ref, y2_ref, w_ref, b_ref, c_ref,
             wci_ref, wcf_ref, wco_ref, h_ref, cn_ref):
    a = jnp.concatenate(
        [x0_ref[0], x0_ref[1], y1_ref[0], y1_ref[1], y2_ref[0], y2_ref[1]],
        axis=1)
    w0, w1, w2 = w_ref[0], w_ref[1], w_ref[2]
    weff = jnp.concatenate([w0 - w2, w1, 2.0 * w2], axis=0)
    conv_t = lax.dot_general(weff, a, (((0,), (1,)), ((), ())),
                             preferred_element_type=jnp.float32)
    conv_t = conv_t + b_ref[0][:, None]
    cc = c_ref[...]
    ig = jax.nn.sigmoid(conv_t[0:16] + wci_ref[...] * cc)
    fg = jax.nn.sigmoid(conv_t[16:32] + wcf_ref[...] * cc)
    c_new = fg * cc + ig * jnp.tanh(conv_t[32:48])
    og = jax.nn.sigmoid(conv_t[48:64] + wco_ref[...] * c_new)
    h_ref[...] = og * jnp.tanh(c_new)
    cn_ref[...] = c_new


_tc_gates = pl.pallas_call(
    _tc_body,
    grid=(GRID,),
    in_specs=[
        pl.BlockSpec((NC, NB, F), lambda i: (0, i, 0)),
        pl.BlockSpec((NC, NB, F), lambda i: (0, i, 0)),
        pl.BlockSpec((NC, NB, F), lambda i: (0, i, 0)),
        pl.BlockSpec((3, 32, 64), lambda i: (0, 0, 0)),
        pl.BlockSpec((1, 64), lambda i: (0, 0)),
        pl.BlockSpec((HID, NB), lambda i: (0, i)),
        pl.BlockSpec((HID, NB), lambda i: (0, i)),
        pl.BlockSpec((HID, NB), lambda i: (0, i)),
        pl.BlockSpec((HID, NB), lambda i: (0, i)),
    ],
    out_specs=[
        pl.BlockSpec((HID, NB), lambda i: (0, i)),
        pl.BlockSpec((HID, NB), lambda i: (0, i)),
    ],
    out_shape=[
        jax.ShapeDtypeStruct((HID, N), jnp.float32),
        jax.ShapeDtypeStruct((HID, N), jnp.float32),
    ],
)


def kernel(input_tensor, h_cur, c_cur, lap_rows, lap_cols, lap_vals,
           W_cheb, b, W_ci, W_cf, W_co):
    x0 = jnp.concatenate(
        [input_tensor[0].T, h_cur[0].T], axis=0)
    cols3 = lap_cols.reshape(NWIN, KR, 128)
    rows3 = lap_rows.reshape(NWIN, KR, 128)
    vals3 = lap_vals.reshape(NWIN, KR, 128)

    sc_lap = _sc_lap()
    y1 = sc_lap(x0, cols3, rows3, vals3)
    y2 = sc_lap(y1, cols3, rows3, vals3)

    h_new, c_new = _tc_gates(
        x0.reshape(NC, N, F), y1.reshape(NC, N, F), y2.reshape(NC, N, F),
        W_cheb, b.reshape(1, 64), c_cur[0], W_ci, W_cf, W_co)
    return h_new[None], c_new[None]
